# baseline (speedup 1.0000x reference)
import functools

import jax
import jax.numpy as jnp
from jax.experimental import pallas as pl

_F32 = jnp.float32


def _conv_body(x_ref, w_ref, s_ref, b_ref, o_ref, *, taps, w_row, m_rows,
               leaky):
    acc = None
    for dy in range(taps):
        xs = x_ref[0, pl.ds(dy * w_row, m_rows), :]
        t = jnp.dot(xs, w_ref[dy], preferred_element_type=_F32)
        acc = t if acc is None else acc + t
    y = acc * s_ref[...] + b_ref[...]
    if leaky:
        y = jnp.where(y > 0, y, _F32(0.1) * y)
    o_ref[0] = y


def _conv(x3d, wtap, scale, bias, *, w_row, h_out, leaky):
    g_num, r_rows, k_dim = x3d.shape
    taps, _, c_out = wtap.shape
    m_rows = h_out * w_row
    assert r_rows == (h_out + taps - 1) * w_row, (x3d.shape, wtap.shape, w_row, h_out)
    body = functools.partial(_conv_body, taps=taps, w_row=w_row,
                             m_rows=m_rows, leaky=leaky)
    return pl.pallas_call(
        body,
        grid=(g_num,),
        in_specs=[
            pl.BlockSpec((1, r_rows, k_dim), lambda g: (g, 0, 0)),
            pl.BlockSpec((taps, k_dim, c_out), lambda g: (0, 0, 0)),
            pl.BlockSpec((1, c_out), lambda g: (0, 0)),
            pl.BlockSpec((1, c_out), lambda g: (0, 0)),
        ],
        out_specs=pl.BlockSpec((1, m_rows, c_out), lambda g: (g, 0, 0)),
        out_shape=jax.ShapeDtypeStruct((g_num, m_rows, c_out), _F32),
    )(x3d, wtap, scale.reshape(1, c_out), bias.reshape(1, c_out))


def _pool_body(x_ref, o5_ref, o9_ref, o13_ref):
    a = x_ref[0]

    def pool5(v):
        h2, w2 = v.shape[0] - 4, v.shape[1] - 4
        r = v[0:h2]
        for i in range(1, 5):
            r = jnp.maximum(r, v[i:i + h2])
        c = r[:, 0:w2]
        for i in range(1, 5):
            c = jnp.maximum(c, r[:, i:i + w2])
        return c

    m5 = pool5(a)
    m9 = pool5(m5)
    m13 = pool5(m9)
    o5_ref[0] = m5[4:17, 4:17]
    o9_ref[0] = m9[2:15, 2:15]
    o13_ref[0] = m13


def _spp_pools(s):
    n, h, w, c = s.shape
    pad = jnp.full((n, h + 12, w + 12, c), _F32(-1e30))
    canvas = pad.at[:, 6:6 + h, 6:6 + w, :].set(s)
    shp = jax.ShapeDtypeStruct((n, h, w, c), _F32)
    return pl.pallas_call(
        _pool_body,
        grid=(n,),
        in_specs=[pl.BlockSpec((1, h + 12, w + 12, c), lambda g: (g, 0, 0, 0))],
        out_specs=[pl.BlockSpec((1, h, w, c), lambda g: (g, 0, 0, 0))] * 3,
        out_shape=[shp, shp, shp],
    )(canvas)


def _fold_s1(x):
    n, h, w, c = x.shape
    xp = jnp.pad(x, ((0, 0), (1, 1), (1, 1), (0, 0)))
    cat = jnp.concatenate([xp[:, :, 0:w], xp[:, :, 1:w + 1], xp[:, :, 2:w + 2]],
                          axis=-1)
    return cat.reshape(n, (h + 2) * w, 3 * c)


def _w_s1(p):
    return jnp.transpose(p, (2, 3, 1, 0)).reshape(3, -1, p.shape[0])


def _fold_s2(x):
    n, h, w, c = x.shape
    hh, ww = h // 2, w // 2
    xp = jnp.pad(x, ((0, 0), (0, 2), (0, 2), (0, 0)))
    s2d = xp.reshape(n, hh + 1, 2, ww + 1, 2, c)
    s2d = jnp.transpose(s2d, (0, 1, 3, 2, 4, 5)).reshape(n, hh + 1, ww + 1, 4 * c)
    cat = jnp.concatenate([s2d[:, :, 0:ww], s2d[:, :, 1:ww + 1]], axis=-1)
    return cat.reshape(n, (hh + 1) * ww, 8 * c)


def _w_s2(p):
    o, c = p.shape[0], p.shape[1]
    wp = jnp.pad(p, ((0, 0), (0, 0), (0, 1), (0, 1)))
    wr = wp.reshape(o, c, 2, 2, 2, 2)
    return jnp.transpose(wr, (2, 4, 3, 5, 1, 0)).reshape(2, 8 * c, o)


def _upsample2(x):
    n, h, w, c = x.shape
    x = jnp.broadcast_to(x[:, :, None, :, None, :], (n, h, 2, w, 2, c))
    return x.reshape(n, 2 * h, 2 * w, c)


def _ones_bias(p):
    return p["scale"], p["bias"]


def _cbl_s2(x, p, *, h_out, w_out):
    return _conv(_fold_s2(x), _w_s2(p["w"]), p["scale"], p["bias"],
                 w_row=w_out, h_out=h_out, leaky=True).reshape(
                     x.shape[0], h_out, w_out, -1)


def _cbl_s1_3x3(x, p):
    n, h, w, _ = x.shape
    return _conv(_fold_s1(x), _w_s1(p["w"]), p["scale"], p["bias"],
                 w_row=w, h_out=h, leaky=True).reshape(n, h, w, -1)


def _pw(x, w_oi11, scale, bias, *, leaky):
    n, h, w, c = x.shape
    wt = w_oi11[:, :, 0, 0].T.reshape(1, c, -1)
    out = _conv(x.reshape(n, h * w, c), wt, scale, bias,
                w_row=h * w, h_out=1, leaky=leaky)
    return out.reshape(n, h, w, -1)


def _cbl_1x1(x, p):
    return _pw(x, p["w"], p["scale"], p["bias"], leaky=True)


def kernel(x, params):
    p = params
    n = x.shape[0]
    xh = jnp.transpose(x, (0, 2, 3, 1))

    xp = jnp.pad(xh, ((0, 0), (0, 2), (0, 2), (0, 0)))
    cols = [xp[:, dy:dy + 416:2, dx:dx + 416:2, :]
            for dy in range(3) for dx in range(3)]
    xb1 = jnp.concatenate(cols, axis=-1).reshape(n * 8, 208 * 208 // 8, 27)
    wb1 = jnp.transpose(p["b1"]["w"], (2, 3, 1, 0)).reshape(1, 27, 32)
    f1 = _conv(xb1, wb1, p["b1"]["scale"], p["b1"]["bias"],
               w_row=208 * 208 // 8, h_out=1, leaky=True)
    f1 = f1.reshape(n, 208, 208, 32)

    f2 = _cbl_s2(f1, p["b2"], h_out=104, w_out=104)
    feat_small = _cbl_s2(f2, p["b3"], h_out=52, w_out=52)
    feat_medium = _cbl_s2(feat_small, p["b4"], h_out=26, w_out=26)
    feat_large = _cbl_s2(feat_medium, p["b5"], h_out=13, w_out=13)

    s = _cbl_1x1(feat_large, p["spp_c1"])
    m5, m9, m13 = _spp_pools(s)
    s_cat = jnp.concatenate([s, m5, m9, m13], axis=-1)
    p5 = _cbl_1x1(s_cat, p["spp_c2"])

    p5_up = _upsample2(_cbl_1x1(p5, p["conv_up1"]))
    p4 = _cbl_1x1(feat_medium, p["lateral1"])
    p4 = jnp.concatenate([p4, p5_up], axis=-1)
    p4 = _cbl_1x1(p4, p["merge1_0"])
    p4 = _cbl_s1_3x3(p4, p["merge1_1"])
    p4 = _cbl_1x1(p4, p["merge1_2"])

    p4_up = _upsample2(_cbl_1x1(p4, p["conv_up2"]))
    p3 = _cbl_1x1(feat_small, p["lateral2"])
    p3 = jnp.concatenate([p3, p4_up], axis=-1)
    p3 = _cbl_1x1(p3, p["merge2_0"])
    p3 = _cbl_s1_3x3(p3, p["merge2_1"])
    p3 = _cbl_1x1(p3, p["merge2_2"])

    def head(feat, p0, p1):
        h = _cbl_s1_3x3(feat, p0)
        c_out = p1["w"].shape[0]
        out = _pw(h, p1["w"], jnp.ones((c_out,), _F32), p1["b"], leaky=False)
        return jnp.transpose(out, (0, 3, 1, 2))

    out_small = head(p3, p["head_s_0"], p["head_s_1"])
    out_medium = head(p4, p["head_m_0"], p["head_m_1"])
    out_large = head(p5, p["head_l_0"], p["head_l_1"])
    return (out_small, out_medium, out_large)
